# speedup vs baseline: 1.9259x; 1.9259x over previous
import functools

import jax
import jax.numpy as jnp
from jax import lax
from jax.experimental import pallas as pl
from jax.experimental.pallas import tpu as pltpu
from jax.experimental.pallas import tpu_sc as plsc

D_MODEL = 1024
EPS = 1e-5

_NUM_WORKERS = 32
_GATHER_CHUNK = 32


def _sc_gather(token_table, ids_flat):
    n = ids_flat.shape[0]
    d = token_table.shape[1]
    b_per_w = n // _NUM_WORKERS
    mesh = plsc.VectorSubcoreMesh(core_axis_name="c", subcore_axis_name="s")

    @functools.partial(
        pl.kernel,
        mesh=mesh,
        out_type=jax.ShapeDtypeStruct((n, d), jnp.float32),
        scratch_types=[
            pltpu.VMEM((_GATHER_CHUNK,), jnp.int32),
            pltpu.VMEM((_GATHER_CHUNK, d), jnp.float32),
            pltpu.SemaphoreType.DMA,
        ],
    )
    def gather_kernel(table_hbm, idx_hbm, out_hbm, idx_v, rows_v, sem):
        wid = lax.axis_index("s") * 2 + lax.axis_index("c")
        base = wid * b_per_w

        @pl.loop(0, b_per_w, step=_GATHER_CHUNK)
        def _(off):
            pltpu.sync_copy(idx_hbm.at[pl.ds(base + off, _GATHER_CHUNK)], idx_v)
            pltpu.async_copy(table_hbm.at[idx_v], rows_v, sem).wait()
            pltpu.sync_copy(rows_v, out_hbm.at[pl.ds(base + off, _GATHER_CHUNK)])

    return gather_kernel(token_table, ids_flat)


def _ln_body(g_ref, pos_ref, segf_ref, segtab_ref, gamma_ref, beta_ref, o_ref):
    x = g_ref[...]
    s0 = segtab_ref[0, :]
    s1 = segtab_ref[1, :]
    x = x + pos_ref[...] + s0[None, :] + segf_ref[...] * (s1 - s0)[None, :]
    mean = jnp.mean(x, axis=-1, keepdims=True)
    xc = x - mean
    var = jnp.mean(xc * xc, axis=-1, keepdims=True)
    o_ref[...] = gamma_ref[...] * (xc * lax.rsqrt(var + EPS)) + beta_ref[...]


def _tc_ln(gathered, pos_table, segf, seg_table, gamma, beta, blk):
    n, d = gathered.shape
    s = pos_table.shape[0]
    pos_blocks = s // blk
    return pl.pallas_call(
        _ln_body,
        grid=(n // blk,),
        in_specs=[
            pl.BlockSpec((blk, d), lambda i: (i, 0)),
            pl.BlockSpec((blk, d), lambda i: (i % pos_blocks, 0)),
            pl.BlockSpec((blk, 1), lambda i: (i, 0)),
            pl.BlockSpec((2, d), lambda i: (0, 0)),
            pl.BlockSpec((1, d), lambda i: (0, 0)),
            pl.BlockSpec((1, d), lambda i: (0, 0)),
        ],
        out_specs=pl.BlockSpec((blk, d), lambda i: (i, 0)),
        out_shape=jax.ShapeDtypeStruct((n, d), jnp.float32),
    )(gathered, pos_table, segf, seg_table, gamma, beta)


def kernel(input_ids, segment_ids, token_table, pos_table, seg_table, gamma, beta):
    b, s = input_ids.shape
    ids_flat = input_ids.reshape(-1).astype(jnp.int32)
    segf = segment_ids.reshape(-1, 1).astype(jnp.float32)

    gathered = _sc_gather(token_table, ids_flat)

    out = _tc_ln(
        gathered,
        pos_table,
        segf,
        seg_table,
        gamma.reshape(1, -1),
        beta.reshape(1, -1),
        blk=256,
    )
    return out.reshape(b, s, D_MODEL)


# speedup vs baseline: 2.0613x; 1.0703x over previous
import functools

import jax
import jax.numpy as jnp
from jax import lax
from jax.experimental import pallas as pl
from jax.experimental.pallas import tpu as pltpu
from jax.experimental.pallas import tpu_sc as plsc

D_MODEL = 1024
EPS = 1e-5

_NUM_WORKERS = 32
_GATHER_CHUNK = 32


def _sc_gather(token_table, ids_flat):
    n = ids_flat.shape[0]
    d = token_table.shape[1]
    b_per_w = n // _NUM_WORKERS
    ch = _GATHER_CHUNK
    n_ch = b_per_w // ch
    nb = 3
    mesh = plsc.VectorSubcoreMesh(core_axis_name="c", subcore_axis_name="s")

    @functools.partial(
        pl.kernel,
        mesh=mesh,
        out_type=jax.ShapeDtypeStruct((n, d), jnp.float32),
        scratch_types=[
            pltpu.VMEM((b_per_w,), jnp.int32),
        ]
        + [pltpu.VMEM((ch, d), jnp.float32) for _ in range(nb)]
        + [pltpu.SemaphoreType.DMA for _ in range(2 * nb)],
    )
    def gather_kernel(table_hbm, idx_hbm, out_hbm, idx_v, *rest):
        bufs = rest[:nb]
        gsems = rest[nb:2 * nb]
        wsems = rest[2 * nb:]
        wid = lax.axis_index("s") * 2 + lax.axis_index("c")
        base = wid * b_per_w
        pltpu.sync_copy(idx_hbm.at[pl.ds(base, b_per_w)], idx_v)

        def start_gather(c):
            cp = pltpu.make_async_copy(
                table_hbm.at[idx_v.at[pl.ds(c * ch, ch)]],
                bufs[c % nb],
                gsems[c % nb],
            )
            cp.start()
            return cp

        def start_wb(c):
            cp = pltpu.make_async_copy(
                bufs[c % nb],
                out_hbm.at[pl.ds(base + c * ch, ch)],
                wsems[c % nb],
            )
            cp.start()
            return cp

        gathers = [None] * n_ch
        wbs = [None] * n_ch
        for c in range(min(nb - 1, n_ch)):
            gathers[c] = start_gather(c)
        for c in range(n_ch):
            gathers[c].wait()
            wbs[c] = start_wb(c)
            j = c + nb - 1
            if j < n_ch:
                if c >= 1:
                    wbs[c - 1].wait()
                gathers[j] = start_gather(j)
        for c in range(max(0, n_ch - nb), n_ch):
            if wbs[c] is not None:
                wbs[c].wait()

    return gather_kernel(token_table, ids_flat)


def _ln_body(g_ref, pos_ref, segf_ref, segtab_ref, gamma_ref, beta_ref, o_ref):
    x = g_ref[...]
    s0 = segtab_ref[0, :]
    s1 = segtab_ref[1, :]
    x = x + pos_ref[...] + s0[None, :] + segf_ref[...] * (s1 - s0)[None, :]
    mean = jnp.mean(x, axis=-1, keepdims=True)
    xc = x - mean
    var = jnp.mean(xc * xc, axis=-1, keepdims=True)
    o_ref[...] = gamma_ref[...] * (xc * lax.rsqrt(var + EPS)) + beta_ref[...]


def _tc_ln(gathered, pos_table, segf, seg_table, gamma, beta, blk):
    n, d = gathered.shape
    s = pos_table.shape[0]
    pos_blocks = s // blk
    return pl.pallas_call(
        _ln_body,
        grid=(n // blk,),
        in_specs=[
            pl.BlockSpec((blk, d), lambda i: (i, 0)),
            pl.BlockSpec((blk, d), lambda i: (i % pos_blocks, 0)),
            pl.BlockSpec((blk, 1), lambda i: (i, 0)),
            pl.BlockSpec((2, d), lambda i: (0, 0)),
            pl.BlockSpec((1, d), lambda i: (0, 0)),
            pl.BlockSpec((1, d), lambda i: (0, 0)),
        ],
        out_specs=pl.BlockSpec((blk, d), lambda i: (i, 0)),
        out_shape=jax.ShapeDtypeStruct((n, d), jnp.float32),
    )(gathered, pos_table, segf, seg_table, gamma, beta)


def kernel(input_ids, segment_ids, token_table, pos_table, seg_table, gamma, beta):
    b, s = input_ids.shape
    ids_flat = input_ids.reshape(-1).astype(jnp.int32)
    segf = segment_ids.reshape(-1, 1).astype(jnp.float32)

    gathered = _sc_gather(token_table, ids_flat)

    out = _tc_ln(
        gathered,
        pos_table,
        segf,
        seg_table,
        gamma.reshape(1, -1),
        beta.reshape(1, -1),
        blk=256,
    )
    return out.reshape(b, s, D_MODEL)


# speedup vs baseline: 2.0983x; 1.0180x over previous
import functools

import jax
import jax.numpy as jnp
from jax import lax
from jax.experimental import pallas as pl
from jax.experimental.pallas import tpu as pltpu
from jax.experimental.pallas import tpu_sc as plsc

D_MODEL = 1024
EPS = 1e-5

_NUM_WORKERS = 32
_GATHER_CHUNK = 32


def _sc_gather(token_table, ids_flat):
    n = ids_flat.shape[0]
    d = token_table.shape[1]
    b_per_w = n // _NUM_WORKERS
    ch = _GATHER_CHUNK
    n_ch = b_per_w // ch
    nb = 3
    mesh = plsc.VectorSubcoreMesh(core_axis_name="c", subcore_axis_name="s")

    @functools.partial(
        pl.kernel,
        mesh=mesh,
        out_type=jax.ShapeDtypeStruct((n, d), jnp.float32),
        scratch_types=[
            pltpu.VMEM((b_per_w,), jnp.int32),
        ]
        + [pltpu.VMEM((ch, d), jnp.float32) for _ in range(nb)]
        + [pltpu.SemaphoreType.DMA for _ in range(2 * nb)],
    )
    def gather_kernel(table_hbm, idx_hbm, out_hbm, idx_v, *rest):
        bufs = rest[:nb]
        gsems = rest[nb:2 * nb]
        wsems = rest[2 * nb:]
        wid = lax.axis_index("s") * 2 + lax.axis_index("c")
        base = wid * b_per_w
        pltpu.sync_copy(idx_hbm.at[pl.ds(base, b_per_w)], idx_v)

        def start_gather(c):
            cp = pltpu.make_async_copy(
                table_hbm.at[idx_v.at[pl.ds(c * ch, ch)]],
                bufs[c % nb],
                gsems[c % nb],
            )
            cp.start()
            return cp

        def start_wb(c):
            cp = pltpu.make_async_copy(
                bufs[c % nb],
                out_hbm.at[pl.ds(base + c * ch, ch)],
                wsems[c % nb],
            )
            cp.start()
            return cp

        gathers = [None] * n_ch
        wbs = [None] * n_ch
        for c in range(min(nb - 1, n_ch)):
            gathers[c] = start_gather(c)
        for c in range(n_ch):
            gathers[c].wait()
            wbs[c] = start_wb(c)
            j = c + nb - 1
            if j < n_ch:
                if c >= 1:
                    wbs[c - 1].wait()
                gathers[j] = start_gather(j)
        for c in range(max(0, n_ch - nb), n_ch):
            if wbs[c] is not None:
                wbs[c].wait()

    return gather_kernel(token_table, ids_flat)


def _ln_body(g_ref, pos_ref, segf_ref, segtab_ref, gamma_ref, beta_ref, o_ref):
    x = g_ref[...]
    s0 = segtab_ref[0, :]
    s1 = segtab_ref[1, :]
    x = x + pos_ref[...] + s0[None, :] + segf_ref[...] * (s1 - s0)[None, :]
    mean = jnp.mean(x, axis=-1, keepdims=True)
    xc = x - mean
    var = jnp.mean(xc * xc, axis=-1, keepdims=True)
    o_ref[...] = gamma_ref[...] * (xc * lax.rsqrt(var + EPS)) + beta_ref[...]


def _tc_ln(gathered, pos_table, segf, seg_table, gamma, beta, blk):
    n, d = gathered.shape
    s = pos_table.shape[0]
    pos_blocks = s // blk
    batches = n // s
    return pl.pallas_call(
        _ln_body,
        grid=(pos_blocks, batches),
        in_specs=[
            pl.BlockSpec((blk, d), lambda j, b: (b * pos_blocks + j, 0)),
            pl.BlockSpec((blk, d), lambda j, b: (j, 0)),
            pl.BlockSpec((blk, 1), lambda j, b: (b * pos_blocks + j, 0)),
            pl.BlockSpec((2, d), lambda j, b: (0, 0)),
            pl.BlockSpec((1, d), lambda j, b: (0, 0)),
            pl.BlockSpec((1, d), lambda j, b: (0, 0)),
        ],
        out_specs=pl.BlockSpec((blk, d), lambda j, b: (b * pos_blocks + j, 0)),
        out_shape=jax.ShapeDtypeStruct((n, d), jnp.float32),
    )(gathered, pos_table, segf, seg_table, gamma, beta)


def kernel(input_ids, segment_ids, token_table, pos_table, seg_table, gamma, beta):
    b, s = input_ids.shape
    ids_flat = input_ids.reshape(-1).astype(jnp.int32)
    segf = segment_ids.reshape(-1, 1).astype(jnp.float32)

    gathered = _sc_gather(token_table, ids_flat)

    out = _tc_ln(
        gathered,
        pos_table,
        segf,
        seg_table,
        gamma.reshape(1, -1),
        beta.reshape(1, -1),
        blk=256,
    )
    return out.reshape(b, s, D_MODEL)


# speedup vs baseline: 2.3359x; 1.1132x over previous
import functools

import jax
import jax.numpy as jnp
from jax import lax
from jax.experimental import pallas as pl
from jax.experimental.pallas import tpu as pltpu
from jax.experimental.pallas import tpu_sc as plsc

D_MODEL = 1024
EPS = 1e-5

_NUM_WORKERS = 32
_GATHER_CHUNK = 32


def _sc_gather(token_table, ids_flat):
    n = ids_flat.shape[0]
    d = token_table.shape[1]
    b_per_w = n // _NUM_WORKERS
    ch = _GATHER_CHUNK
    n_ch = b_per_w // ch
    nb = 3
    mesh = plsc.VectorSubcoreMesh(core_axis_name="c", subcore_axis_name="s")

    @functools.partial(
        pl.kernel,
        mesh=mesh,
        out_type=jax.ShapeDtypeStruct((n, d), jnp.float32),
        scratch_types=[
            pltpu.VMEM((b_per_w,), jnp.int32),
        ]
        + [pltpu.VMEM((ch, d), jnp.float32) for _ in range(nb)]
        + [pltpu.SemaphoreType.DMA for _ in range(2 * nb)],
    )
    def gather_kernel(table_hbm, idx_hbm, out_hbm, idx_v, *rest):
        bufs = rest[:nb]
        gsems = rest[nb:2 * nb]
        wsems = rest[2 * nb:]
        wid = lax.axis_index("s") * 2 + lax.axis_index("c")
        base = wid * b_per_w
        pltpu.sync_copy(idx_hbm.at[pl.ds(base, b_per_w)], idx_v)

        def start_gather(c):
            cp = pltpu.make_async_copy(
                table_hbm.at[idx_v.at[pl.ds(c * ch, ch)]],
                bufs[c % nb],
                gsems[c % nb],
            )
            cp.start()
            return cp

        def start_wb(c):
            cp = pltpu.make_async_copy(
                bufs[c % nb],
                out_hbm.at[pl.ds(base + c * ch, ch)],
                wsems[c % nb],
            )
            cp.start()
            return cp

        gathers = [None] * n_ch
        wbs = [None] * n_ch
        for c in range(min(nb - 1, n_ch)):
            gathers[c] = start_gather(c)
        for c in range(n_ch):
            gathers[c].wait()
            wbs[c] = start_wb(c)
            j = c + nb - 1
            if j < n_ch:
                if c >= 1:
                    wbs[c - 1].wait()
                gathers[j] = start_gather(j)
        for c in range(max(0, n_ch - nb), n_ch):
            if wbs[c] is not None:
                wbs[c].wait()

    return gather_kernel(token_table, ids_flat)


def _ln_body(g_ref, pos_ref, segf_ref, segtab_ref, gamma_ref, beta_ref, o_ref):
    x = g_ref[...]
    s0 = segtab_ref[0, :]
    s1 = segtab_ref[1, :]
    x = x + pos_ref[...] + s0[None, :] + segf_ref[...] * (s1 - s0)[None, :]
    mean = jnp.mean(x, axis=-1, keepdims=True)
    xc = x - mean
    var = jnp.mean(xc * xc, axis=-1, keepdims=True)
    o_ref[...] = gamma_ref[...] * (xc * lax.rsqrt(var + EPS)) + beta_ref[...]


def _tc_ln(gathered, pos_table, segf, seg_table, gamma, beta, blk):
    n, d = gathered.shape
    s = pos_table.shape[0]
    pos_blocks = s // blk
    batches = n // s
    return pl.pallas_call(
        _ln_body,
        grid=(pos_blocks, batches),
        in_specs=[
            pl.BlockSpec((blk, d), lambda j, b: (b * pos_blocks + j, 0)),
            pl.BlockSpec((blk, d), lambda j, b: (j, 0)),
            pl.BlockSpec((blk, 1), lambda j, b: (b * pos_blocks + j, 0)),
            pl.BlockSpec((2, d), lambda j, b: (0, 0)),
            pl.BlockSpec((1, d), lambda j, b: (0, 0)),
            pl.BlockSpec((1, d), lambda j, b: (0, 0)),
        ],
        out_specs=pl.BlockSpec((blk, d), lambda j, b: (b * pos_blocks + j, 0)),
        out_shape=jax.ShapeDtypeStruct((n, d), jnp.float32),
    )(gathered, pos_table, segf, seg_table, gamma, beta)


def kernel(input_ids, segment_ids, token_table, pos_table, seg_table, gamma, beta):
    b, s = input_ids.shape
    ids_flat = input_ids.reshape(-1).astype(jnp.int32)
    segf = segment_ids.reshape(-1, 1).astype(jnp.float32)

    gathered = _sc_gather(token_table, ids_flat)

    out = _tc_ln(
        gathered,
        pos_table,
        segf,
        seg_table,
        gamma.reshape(1, -1),
        beta.reshape(1, -1),
        blk=512,
    )
    return out.reshape(b, s, D_MODEL)
